# baseline (speedup 1.0000x reference)
import functools

import jax
import jax.numpy as jnp
from jax import lax
from jax.experimental import pallas as pl
from jax.experimental.pallas import tpu as pltpu
from jax.experimental.pallas import tpu_sc as plsc

N = 10000
E = 320000
F = 128
NC = 2
NS = 16
NW = NC * NS
B = 128
RPW = 80
EPAD = NW * RPW * B
ROWS = EPAD // B
RPT = ROWS // NS
NPAD = 10240
SLICE = NPAD // NS
BLK = 8
NBLK = RPW // BLK
G = 16
NG_FULL = SLICE // G
NG_LAST = (N - (NS - 1) * SLICE) // G

_MESH = plsc.VectorSubcoreMesh(core_axis_name="c", subcore_axis_name="s")


@functools.partial(
    pl.kernel,
    out_type=jax.ShapeDtypeStruct((NPAD,), jnp.float32),
    mesh=_MESH,
    scratch_types=[
        pltpu.VMEM((RPT, B), jnp.int32),
        pltpu.VMEM((RPT, B), jnp.float32),
        pltpu.VMEM((SLICE,), jnp.float32),
        pltpu.VMEM_SHARED((NPAD,), jnp.float32),
    ],
)
def _deg_kernel(dst_hbm, ew_hbm, deg_hbm, idx_v, val_v, work_v, acc):
    c = lax.axis_index("c")
    s = lax.axis_index("s")
    zero16 = jnp.zeros((16,), jnp.float32)

    def zput(i, _):
        work_v[pl.ds(i * 16, 16)] = zero16
        return _

    lax.fori_loop(0, SLICE // 16, zput, None)
    pltpu.sync_copy(work_v, acc.at[pl.ds(s * SLICE, SLICE)])
    plsc.subcore_barrier()

    pltpu.sync_copy(dst_hbm.at[s], idx_v)
    pltpu.sync_copy(ew_hbm.at[s], val_v)

    def scadd(ci, _):
        pltpu.sync_copy(val_v.at[ci], acc.at[idx_v.at[ci]], add=True)
        return _

    lax.fori_loop(0, RPT, scadd, None)
    plsc.subcore_barrier()

    @pl.when(c == 0)
    def _():
        pltpu.sync_copy(acc.at[pl.ds(s * SLICE, SLICE)],
                        deg_hbm.at[pl.ds(s * SLICE, SLICE)])


@functools.partial(
    pl.kernel,
    out_type=jax.ShapeDtypeStruct((NW, RPW, B), jnp.float32),
    mesh=_MESH,
    scratch_types=[
        pltpu.VMEM((RPW, B), jnp.int32),
        pltpu.VMEM((RPW, B), jnp.int32),
        pltpu.VMEM((RPW, B), jnp.float32),
        pltpu.VMEM((RPW, B), jnp.float32),
        pltpu.VMEM((2, B), jnp.float32),
        pltpu.VMEM((2, B), jnp.float32),
        pltpu.SemaphoreType.DMA,
        pltpu.SemaphoreType.DMA,
    ],
)
def _norm_kernel(src_hbm, dst_hbm, ew_hbm, dis_hbm, norm_hbm, sv, dv, wv,
                 nv, a_v, b_v, sem0, sem1):
    c = lax.axis_index("c")
    s = lax.axis_index("s")
    w = c * NS + s
    pltpu.sync_copy(src_hbm.at[w], sv)
    pltpu.sync_copy(dst_hbm.at[w], dv)
    pltpu.sync_copy(ew_hbm.at[w], wv)

    def gath(r, p, sem):
        pltpu.async_copy(dis_hbm.at[sv.at[r]], a_v.at[p], sem)
        pltpu.async_copy(dis_hbm.at[dv.at[r]], b_v.at[p], sem)

    def wait(r, p, sem):
        pltpu.make_async_copy(dis_hbm.at[sv.at[r]], a_v.at[p], sem).wait()
        pltpu.make_async_copy(dis_hbm.at[dv.at[r]], b_v.at[p], sem).wait()

    def scale(r, p):
        for j in range(B // 16):
            sl = pl.ds(j * 16, 16)
            nv[r, sl] = a_v[p, sl] * wv[r, sl] * b_v[p, sl]

    gath(0, 0, sem0)
    gath(1, 1, sem1)

    def pair(t, _):
        ra = 2 * t
        rb = 2 * t + 1
        wait(ra, 0, sem0)
        scale(ra, 0)

        @pl.when(t < RPW // 2 - 1)
        def _():
            gath(ra + 2, 0, sem0)

        wait(rb, 1, sem1)
        scale(rb, 1)

        @pl.when(t < RPW // 2 - 1)
        def _():
            gath(rb + 2, 1, sem1)

        return _

    lax.fori_loop(0, RPW // 2, pair, None)
    pltpu.sync_copy(nv, norm_hbm.at[w])


@functools.partial(
    pl.kernel,
    out_type=jax.ShapeDtypeStruct((NC, N, F), jnp.float32),
    mesh=_MESH,
    scratch_types=[
        pltpu.VMEM((RPW, B), jnp.int32),
        pltpu.VMEM((RPW, B), jnp.int32),
        pltpu.VMEM((RPW, B), jnp.float32),
        pltpu.VMEM((B, F), jnp.float32),
        pltpu.VMEM((G, F), jnp.float32),
        pltpu.SemaphoreType.DMA,
        pltpu.SemaphoreType.DMA,
        pltpu.VMEM_SHARED((N, F), jnp.float32),
    ],
)
def _msg_kernel(tab_hbm, src_hbm, dst_hbm, norm_hbm, out_hbm,
                sv, dv, nv, rows_v, zb, sem_z, sem_g, acc):
    c = lax.axis_index("c")
    s = lax.axis_index("s")
    w = c * NS + s
    ngroups = jnp.where(s == NS - 1, NG_LAST, NG_FULL)

    zero16 = jnp.zeros((16,), jnp.float32)
    for k in range(G):
        for j in range(F // 16):
            zb[k, pl.ds(j * 16, 16)] = zero16

    def zput(g, _):
        pltpu.sync_copy(zb, acc.at[pl.ds(s * SLICE + g * G, G)])
        return _

    lax.fori_loop(0, ngroups, zput, None)
    pltpu.sync_copy(src_hbm.at[w], sv)
    pltpu.sync_copy(dst_hbm.at[w], dv)
    pltpu.sync_copy(norm_hbm.at[w], nv)

    def touch(q, _):
        row0 = jnp.minimum(s * SLICE + q * B, N - B)
        pltpu.sync_copy(tab_hbm.at[pl.ds(row0, B)], rows_v)
        return _

    lax.fori_loop(0, SLICE // B, touch, None)
    plsc.subcore_barrier()

    def chunk(ci, _):
        pltpu.async_copy(tab_hbm.at[sv.at[ci]], rows_v, sem_g).wait()
        for jg in range(B // 16):
            v = nv[ci, pl.ds(jg * 16, 16)]
            for k in range(16):
                sc = v[k]
                b = jg * 16 + k
                for j in range(F // 16):
                    rows_v[b, pl.ds(j * 16, 16)] = (
                        rows_v[b, pl.ds(j * 16, 16)] * sc)
        pltpu.sync_copy(rows_v, acc.at[dv.at[ci]], add=True)
        return _

    lax.fori_loop(0, RPW, chunk, None)
    plsc.subcore_barrier()

    @pl.when(s < NS - 1)
    def _():
        pltpu.sync_copy(acc.at[pl.ds(s * SLICE, SLICE)],
                        out_hbm.at[c, pl.ds(s * SLICE, SLICE)])

    @pl.when(s == NS - 1)
    def _():
        last = N - (NS - 1) * SLICE
        pltpu.sync_copy(acc.at[pl.ds((NS - 1) * SLICE, last)],
                        out_hbm.at[c, pl.ds((NS - 1) * SLICE, last)])


_RB = 2000


def _dis_tc(deg_raw):
    def body(d_ref, dis_ref, sn_ref):
        deg = d_ref[...] + 1.0
        y = jnp.where(deg > 0.0, lax.rsqrt(jnp.maximum(deg, 1e-12)), 0.0)
        dis_ref[...] = y
        sn_ref[...] = y * y

    return pl.pallas_call(
        body,
        out_shape=(
            jax.ShapeDtypeStruct((NPAD // F, F), jnp.float32),
            jax.ShapeDtypeStruct((NPAD // F, F), jnp.float32),
        ),
    )(deg_raw)


def _xw1_tc(x, W1):
    def body(x_ref, w_ref, o_ref):
        o_ref[...] = jnp.dot(x_ref[...], w_ref[...],
                             preferred_element_type=jnp.float32)

    return pl.pallas_call(
        body,
        grid=(N // _RB,),
        in_specs=[
            pl.BlockSpec((_RB, F), lambda i: (i, 0)),
            pl.BlockSpec((F, F), lambda i: (0, 0)),
        ],
        out_specs=pl.BlockSpec((_RB, F), lambda i: (i, 0)),
        out_shape=jax.ShapeDtypeStruct((N, F), jnp.float32),
    )(x, W1)


def _layer2_tc(P, xw1, sn_col, b1, W2):
    def body(p_ref, x_ref, s_ref, b_ref, w_ref, o_ref):
        h = p_ref[0] + p_ref[1] + x_ref[...] * s_ref[...] + b_ref[...]
        h = jnp.maximum(h, 0.0)
        o_ref[...] = jnp.dot(h, w_ref[...],
                             preferred_element_type=jnp.float32)

    return pl.pallas_call(
        body,
        grid=(N // _RB,),
        in_specs=[
            pl.BlockSpec((NC, _RB, F), lambda i: (0, i, 0)),
            pl.BlockSpec((_RB, F), lambda i: (i, 0)),
            pl.BlockSpec((_RB, 1), lambda i: (i, 0)),
            pl.BlockSpec((F,), lambda i: (0,)),
            pl.BlockSpec((F, F), lambda i: (0, 0)),
        ],
        out_specs=pl.BlockSpec((_RB, F), lambda i: (i, 0)),
        out_shape=jax.ShapeDtypeStruct((N, F), jnp.float32),
    )(P, xw1, sn_col, b1, W2)


def _final_tc(Q, xw2, sn_col, b2):
    def body(q_ref, x_ref, s_ref, b_ref, o_ref):
        o_ref[...] = (q_ref[0] + q_ref[1] + x_ref[...] * s_ref[...]
                      + b_ref[...])

    return pl.pallas_call(
        body,
        grid=(N // _RB,),
        in_specs=[
            pl.BlockSpec((NC, _RB, F), lambda i: (0, i, 0)),
            pl.BlockSpec((_RB, F), lambda i: (i, 0)),
            pl.BlockSpec((_RB, 1), lambda i: (i, 0)),
            pl.BlockSpec((F,), lambda i: (0,)),
        ],
        out_specs=pl.BlockSpec((_RB, F), lambda i: (i, 0)),
        out_shape=jax.ShapeDtypeStruct((N, F), jnp.float32),
    )(Q, xw2, sn_col, b2)


def kernel(x, edge_index, edge_attr, u, batch, W1, b1, W2, b2):
    pad = EPAD - E
    zpad_i = jnp.zeros((pad,), jnp.int32)
    zpad_f = jnp.zeros((pad,), jnp.float32)
    src_f = jnp.concatenate([edge_index[0], zpad_i])
    dst_f = jnp.concatenate([edge_index[1], zpad_i])
    ew_f = jnp.concatenate([edge_attr[:, 0], zpad_f])
    src = src_f.reshape(NW, RPW, B)
    dst = dst_f.reshape(NW, RPW, B)
    ew = ew_f.reshape(NW, RPW, B)
    dst_t = dst_f.reshape(NS, RPT, B)
    ew_t = ew_f.reshape(NS, RPT, B)
    src_m = src_f.reshape(NW, NBLK, BLK * B)
    dst_m = dst_f.reshape(NW, NBLK, BLK, B)

    deg_raw = _deg_kernel(dst_t, ew_t)
    dis2d, sn2d = _dis_tc(deg_raw.reshape(NPAD // F, F))
    dis = dis2d.reshape(NPAD)
    sn_col = sn2d.reshape(NPAD)[:N, None]
    norm = _norm_kernel(src, dst, ew, dis)

    xw1 = _xw1_tc(x, W1)
    P = _msg_kernel(xw1, src, dst, norm)
    xw2 = _layer2_tc(P, xw1, sn_col, b1, W2)
    Q = _msg_kernel(xw2, src, dst, norm)
    return _final_tc(Q, xw2, sn_col, b2)


# speedup vs baseline: 1.5166x; 1.5166x over previous
import functools

import jax
import jax.numpy as jnp
from jax import lax
from jax.experimental import pallas as pl
from jax.experimental.pallas import tpu as pltpu
from jax.experimental.pallas import tpu_sc as plsc

N = 10000
E = 320000
F = 128
NC = 2
NS = 16
NW = NC * NS
B = 128
EPAD = 323584
ROWS = EPAD // B
RPW = ROWS // NW
RPT = ROWS // NS
NPAD = 10240
SLICE = NPAD // NS
G = 16
NG_FULL = SLICE // G
NG_LAST = (N - (NS - 1) * SLICE) // G

_MESH = plsc.VectorSubcoreMesh(core_axis_name="c", subcore_axis_name="s")


@functools.partial(
    pl.kernel,
    out_type=jax.ShapeDtypeStruct((NPAD,), jnp.float32),
    mesh=_MESH,
    scratch_types=[
        pltpu.VMEM((RPT, B), jnp.int32),
        pltpu.VMEM((RPT, B), jnp.float32),
        pltpu.VMEM((SLICE,), jnp.float32),
        pltpu.VMEM_SHARED((NPAD,), jnp.float32),
    ],
)
def _deg_kernel(dst_hbm, ew_hbm, deg_hbm, idx_v, val_v, work_v, acc):
    c = lax.axis_index("c")
    s = lax.axis_index("s")
    zero16 = jnp.zeros((16,), jnp.float32)

    def zput(i, _):
        work_v[pl.ds(i * 16, 16)] = zero16
        return _

    lax.fori_loop(0, SLICE // 16, zput, None)
    pltpu.sync_copy(work_v, acc.at[pl.ds(s * SLICE, SLICE)])
    plsc.subcore_barrier()

    pltpu.sync_copy(dst_hbm.at[s], idx_v)
    pltpu.sync_copy(ew_hbm.at[s], val_v)

    def scadd(ci, _):
        pltpu.sync_copy(val_v.at[ci], acc.at[idx_v.at[ci]], add=True)
        return _

    lax.fori_loop(0, RPT, scadd, None)
    plsc.subcore_barrier()

    @pl.when(c == 0)
    def _():
        pltpu.sync_copy(acc.at[pl.ds(s * SLICE, SLICE)],
                        deg_hbm.at[pl.ds(s * SLICE, SLICE)])


@functools.partial(
    pl.kernel,
    out_type=jax.ShapeDtypeStruct((NW, RPW, B), jnp.float32),
    mesh=_MESH,
    scratch_types=[
        pltpu.VMEM((RPW, B), jnp.int32),
        pltpu.VMEM((RPW, B), jnp.int32),
        pltpu.VMEM((RPW, B), jnp.float32),
        pltpu.VMEM((RPW, B), jnp.float32),
        pltpu.VMEM((B,), jnp.float32),
        pltpu.VMEM((B,), jnp.float32),
        pltpu.SemaphoreType.DMA,
        pltpu.SemaphoreType.DMA,
    ],
)
def _norm_kernel(src_hbm, dst_hbm, ew_hbm, dis_hbm, norm_hbm, sv, dv, wv,
                 nv, a_v, b_v, sem_a, sem_b):
    c = lax.axis_index("c")
    s = lax.axis_index("s")
    w = c * NS + s
    pltpu.sync_copy(src_hbm.at[w], sv)
    pltpu.sync_copy(dst_hbm.at[w], dv)
    pltpu.sync_copy(ew_hbm.at[w], wv)

    def row(r, _):
        da = pltpu.async_copy(dis_hbm.at[sv.at[r]], a_v, sem_a)
        db = pltpu.async_copy(dis_hbm.at[dv.at[r]], b_v, sem_b)
        da.wait()
        db.wait()
        for j in range(B // 16):
            sl = pl.ds(j * 16, 16)
            nv[r, sl] = a_v[sl] * wv[r, sl] * b_v[sl]
        return _

    lax.fori_loop(0, RPW, row, None)
    pltpu.sync_copy(nv, norm_hbm.at[w])


@functools.partial(
    pl.kernel,
    out_type=jax.ShapeDtypeStruct((NC, N, F), jnp.float32),
    mesh=_MESH,
    scratch_types=[
        pltpu.VMEM((RPW, B), jnp.int32),
        pltpu.VMEM((RPW, B), jnp.int32),
        pltpu.VMEM((RPW, B), jnp.float32),
        pltpu.VMEM((B, F), jnp.float32),
        pltpu.VMEM((G, F), jnp.float32),
        pltpu.VMEM((SLICE,), jnp.float32),
        pltpu.VMEM_SHARED((N, F), jnp.float32),
        pltpu.SemaphoreType.DMA,
    ],
)
def _msg_kernel(tab_hbm, src_hbm, dst_hbm, norm_hbm, sn_hbm, out_hbm,
                sv, dv, nv, rows_v, xr_v, sn_t, acc, sem):
    c = lax.axis_index("c")
    s = lax.axis_index("s")
    w = c * NS + s
    ngroups = jnp.where(s == NS - 1, NG_LAST, NG_FULL)

    @pl.when(c == 0)
    def _():
        pltpu.sync_copy(sn_hbm.at[pl.ds(s * SLICE, SLICE)], sn_t)

        def init_group(g, _):
            base = s * SLICE + g * G
            pltpu.sync_copy(tab_hbm.at[c, pl.ds(base, G)], xr_v)
            v = sn_t[pl.ds(g * G, G)]
            for k in range(G):
                sc = v[k]
                for j in range(F // 16):
                    xr_v[k, pl.ds(j * 16, 16)] = (
                        xr_v[k, pl.ds(j * 16, 16)] * sc)
            pltpu.sync_copy(xr_v, acc.at[pl.ds(base, G)])
            return _

        lax.fori_loop(0, ngroups, init_group, None)

    @pl.when(c != 0)
    def _():
        zero16 = jnp.zeros((16,), jnp.float32)
        for k in range(G):
            for j in range(F // 16):
                xr_v[k, pl.ds(j * 16, 16)] = zero16

        def zgroup(g, _):
            pltpu.sync_copy(xr_v, acc.at[pl.ds(s * SLICE + g * G, G)])
            return _

        lax.fori_loop(0, ngroups, zgroup, None)

    plsc.subcore_barrier()

    pltpu.sync_copy(src_hbm.at[w], sv)
    pltpu.sync_copy(dst_hbm.at[w], dv)
    pltpu.sync_copy(norm_hbm.at[w], nv)

    def chunk(ci, _):
        pltpu.async_copy(tab_hbm.at[c].at[sv.at[ci]], rows_v, sem).wait()
        for jg in range(B // 16):
            v = nv[ci, pl.ds(jg * 16, 16)]
            for k in range(16):
                sc = v[k]
                b = jg * 16 + k
                for j in range(F // 16):
                    rows_v[b, pl.ds(j * 16, 16)] = (
                        rows_v[b, pl.ds(j * 16, 16)] * sc)
        pltpu.sync_copy(rows_v, acc.at[dv.at[ci]], add=True)
        return _

    lax.fori_loop(0, RPW, chunk, None)
    plsc.subcore_barrier()

    @pl.when(s < NS - 1)
    def _():
        pltpu.sync_copy(acc.at[pl.ds(s * SLICE, SLICE)],
                        out_hbm.at[c, pl.ds(s * SLICE, SLICE)])

    @pl.when(s == NS - 1)
    def _():
        last = N - (NS - 1) * SLICE
        pltpu.sync_copy(acc.at[pl.ds((NS - 1) * SLICE, last)],
                        out_hbm.at[c, pl.ds((NS - 1) * SLICE, last)])


_RB = 2000


def _dis_tc(deg_raw):
    def body(d_ref, dis_ref, sn_ref):
        deg = d_ref[...] + 1.0
        y = jnp.where(deg > 0.0, lax.rsqrt(jnp.maximum(deg, 1e-12)), 0.0)
        dis_ref[...] = y
        sn_ref[...] = y * y

    return pl.pallas_call(
        body,
        out_shape=(
            jax.ShapeDtypeStruct((NPAD // F, F), jnp.float32),
            jax.ShapeDtypeStruct((NPAD // F, F), jnp.float32),
        ),
    )(deg_raw)


def _xw1_tc(x, W1):
    def body(x_ref, w_ref, o_ref):
        o_ref[...] = jnp.dot(x_ref[...], w_ref[...],
                             preferred_element_type=jnp.float32)

    return pl.pallas_call(
        body,
        grid=(N // _RB,),
        in_specs=[
            pl.BlockSpec((_RB, F), lambda i: (i, 0)),
            pl.BlockSpec((F, F), lambda i: (0, 0)),
        ],
        out_specs=pl.BlockSpec((_RB, F), lambda i: (i, 0)),
        out_shape=jax.ShapeDtypeStruct((N, F), jnp.float32),
    )(x, W1)


def _layer2_tc(P, b1, W2):
    def body(p_ref, b_ref, w_ref, o_ref):
        h = jnp.maximum(p_ref[0] + p_ref[1] + b_ref[...], 0.0)
        o_ref[...] = jnp.dot(h, w_ref[...],
                             preferred_element_type=jnp.float32)

    return pl.pallas_call(
        body,
        grid=(N // _RB,),
        in_specs=[
            pl.BlockSpec((NC, _RB, F), lambda i: (0, i, 0)),
            pl.BlockSpec((F,), lambda i: (0,)),
            pl.BlockSpec((F, F), lambda i: (0, 0)),
        ],
        out_specs=pl.BlockSpec((_RB, F), lambda i: (i, 0)),
        out_shape=jax.ShapeDtypeStruct((N, F), jnp.float32),
    )(P, b1, W2)


def _final_tc(Q, b2):
    def body(q_ref, b_ref, o_ref):
        o_ref[...] = q_ref[0] + q_ref[1] + b_ref[...]

    return pl.pallas_call(
        body,
        grid=(N // _RB,),
        in_specs=[
            pl.BlockSpec((NC, _RB, F), lambda i: (0, i, 0)),
            pl.BlockSpec((F,), lambda i: (0,)),
        ],
        out_specs=pl.BlockSpec((_RB, F), lambda i: (i, 0)),
        out_shape=jax.ShapeDtypeStruct((N, F), jnp.float32),
    )(Q, b2)


def kernel(x, edge_index, edge_attr, u, batch, W1, b1, W2, b2):
    pad = EPAD - E
    zpad_i = jnp.zeros((pad,), jnp.int32)
    zpad_f = jnp.zeros((pad,), jnp.float32)
    src_f = jnp.concatenate([edge_index[0], zpad_i])
    dst_f = jnp.concatenate([edge_index[1], zpad_i])
    ew_f = jnp.concatenate([edge_attr[:, 0], zpad_f])
    src = src_f.reshape(NW, RPW, B)
    dst = dst_f.reshape(NW, RPW, B)
    ew = ew_f.reshape(NW, RPW, B)
    dst_t = dst_f.reshape(NS, RPT, B)
    ew_t = ew_f.reshape(NS, RPT, B)

    deg_raw = _deg_kernel(dst_t, ew_t)
    dis2d, sn2d = _dis_tc(deg_raw.reshape(NPAD // F, F))
    dis = dis2d.reshape(NPAD)
    sn = sn2d.reshape(NPAD)
    norm = _norm_kernel(src, dst, ew, dis)

    xw1 = _xw1_tc(x, W1)
    xw1d = jnp.tile(xw1[None], (NC, 1, 1))
    P = _msg_kernel(xw1d, src, dst, norm, sn)
    xw2 = _layer2_tc(P, b1, W2)
    xw2d = jnp.tile(xw2[None], (NC, 1, 1))
    Q = _msg_kernel(xw2d, src, dst, norm, sn)
    return _final_tc(Q, b2)
